# baseline (speedup 1.0000x reference)
import functools

import jax
import jax.numpy as jnp
from jax import lax
from jax.experimental import pallas as pl
from jax.experimental.pallas import tpu as pltpu
from jax.experimental.pallas import tpu_sc as plsc

_B, _T, _D = 4096, 200, 64
_N = _B * _T
_NC, _NS = 2, 16
_NW = _NC * _NS
_RPW = _N // _NW
_CH = 128
_NCH = _RPW // _CH
_NBUF = 8
_NG = _NCH // _NBUF


def _body(x_hbm, table_hbm, out_hbm, idx_v, rows_v, *sems):
    gsem = sems[:_NBUF]
    wsem = sems[_NBUF:]
    wid = lax.axis_index("s") * _NC + lax.axis_index("c")
    base_w = wid * _RPW
    pltpu.sync_copy(x_hbm.at[wid], idx_v)

    def gather(c, b):
        src = (base_w + c * _CH) % (100000 - _CH)
        pltpu.async_copy(table_hbm.at[pl.ds(src, _CH)], rows_v.at[b], gsem[b])

    def gather_wait(c, b):
        src = (base_w + c * _CH) % (100000 - _CH)
        pltpu.make_async_copy(
            table_hbm.at[pl.ds(src, _CH)], rows_v.at[b], gsem[b]).wait()

    def write(c, b):
        pltpu.async_copy(
            rows_v.at[b], out_hbm.at[pl.ds(base_w + c * _CH, _CH)], wsem[b])

    def write_wait(c, b):
        pltpu.make_async_copy(
            rows_v.at[b], out_hbm.at[pl.ds(base_w + c * _CH, _CH)],
            wsem[b]).wait()

    for b in range(_NBUF):
        gather(b, b)

    def group(g, carry):
        c0 = g * _NBUF
        for b in range(_NBUF):
            gather_wait(c0 + b, b)
            write(c0 + b, b)

        @pl.when(g + 1 < _NG)
        def _():
            for b in range(_NBUF):
                write_wait(c0 + b, b)
                gather(c0 + _NBUF + b, b)

        return carry

    lax.fori_loop(0, _NG, group, 0)

    for b in range(_NBUF):
        write_wait((_NG - 1) * _NBUF + b, b)


@jax.jit
def kernel(x, cluster_centers):
    xw = x.reshape(_NW, _NCH, _CH)
    out = pl.kernel(
        _body,
        out_type=jax.ShapeDtypeStruct((_N, _D), jnp.float32),
        mesh=plsc.VectorSubcoreMesh(core_axis_name="c", subcore_axis_name="s"),
        compiler_params=pltpu.CompilerParams(use_tc_tiling_on_sc=False),
        scratch_types=[
            pltpu.VMEM((_NCH, _CH), jnp.int32),
            pltpu.VMEM((_NBUF, _CH, _D), jnp.float32),
        ] + [pltpu.SemaphoreType.DMA] * (2 * _NBUF),
    )(xw, cluster_centers)
    return out.reshape(_B, _T, _D)


# speedup vs baseline: 1.0051x; 1.0051x over previous
import functools

import jax
import jax.numpy as jnp
from jax import lax
from jax.experimental import pallas as pl
from jax.experimental.pallas import tpu as pltpu
from jax.experimental.pallas import tpu_sc as plsc

_B, _T, _D = 4096, 200, 64
_N = _B * _T
_NC, _NS = 2, 16
_NW = _NC * _NS
_RPW = _N // _NW
_CH = 128
_NCH = _RPW // _CH


_NBUF = 8
_NG = _NCH // _NBUF


def _gather_body(x_hbm, table_hbm, out_hbm, idx_v, rows_v, *sems):
    gsem = sems[:_NBUF]
    wsem = sems[_NBUF:]
    wid = lax.axis_index("s") * _NC + lax.axis_index("c")
    base_w = wid * _RPW
    pltpu.sync_copy(x_hbm.at[wid], idx_v)

    def gather(c, b):
        pltpu.async_copy(table_hbm.at[idx_v.at[c]], rows_v.at[b], gsem[b])

    def gather_wait(c, b):
        pltpu.make_async_copy(
            table_hbm.at[idx_v.at[c]], rows_v.at[b], gsem[b]).wait()

    def write(c, b):
        pltpu.async_copy(
            rows_v.at[b], out_hbm.at[pl.ds(base_w + c * _CH, _CH)], wsem[b])

    def write_wait(c, b):
        pltpu.make_async_copy(
            rows_v.at[b], out_hbm.at[pl.ds(base_w + c * _CH, _CH)],
            wsem[b]).wait()

    for b in range(_NBUF):
        gather(b, b)

    def group(g, carry):
        c0 = g * _NBUF
        for b in range(_NBUF):
            gather_wait(c0 + b, b)
            write(c0 + b, b)

        @pl.when(g + 1 < _NG)
        def _():
            for b in range(_NBUF):
                write_wait(c0 + b, b)
                gather(c0 + _NBUF + b, b)

        return carry

    lax.fori_loop(0, _NG, group, 0)

    for b in range(_NBUF):
        write_wait((_NG - 1) * _NBUF + b, b)


@jax.jit
def kernel(x, cluster_centers):
    xw = x.reshape(_NW, _NCH, _CH)
    out = pl.kernel(
        _gather_body,
        out_type=jax.ShapeDtypeStruct((_N, _D), jnp.float32),
        mesh=plsc.VectorSubcoreMesh(core_axis_name="c", subcore_axis_name="s"),
        compiler_params=pltpu.CompilerParams(use_tc_tiling_on_sc=False),
        scratch_types=[
            pltpu.VMEM((_NCH, _CH), jnp.int32),
            pltpu.VMEM((_NBUF, _CH, _D), jnp.float32),
        ] + [pltpu.SemaphoreType.DMA] * (2 * _NBUF),
    )(xw, cluster_centers)
    return out.reshape(_B, _T, _D)


# speedup vs baseline: 1.0077x; 1.0026x over previous
import jax
import jax.numpy as jnp
from jax import lax
from jax.experimental import pallas as pl
from jax.experimental.pallas import tpu as pltpu
from jax.experimental.pallas import tpu_sc as plsc

_B, _T, _D = 4096, 200, 64
_N = _B * _T
_NC, _NS = 2, 16
_NW = _NC * _NS
_RPW = _N // _NW
_CH = 128
_NCH = _RPW // _CH


_NBUF = 8
_NG = _NCH // _NBUF


def _gather_body(x_hbm, table_hbm, out_hbm, idx_v, rows_v, *sems):
    gsem = sems[:_NBUF]
    wsem = sems[_NBUF:]
    wid = lax.axis_index("s") * _NC + lax.axis_index("c")
    base_w = wid * _RPW
    pltpu.sync_copy(x_hbm.at[wid], idx_v)

    def gather(c, b):
        pltpu.async_copy(table_hbm.at[idx_v.at[c]], rows_v.at[b], gsem[b])

    def gather_wait(c, b):
        pltpu.make_async_copy(
            table_hbm.at[idx_v.at[c]], rows_v.at[b], gsem[b]).wait()

    def write(c, b):
        pltpu.async_copy(
            rows_v.at[b], out_hbm.at[pl.ds(base_w + c * _CH, _CH)], wsem[b])

    def write_wait(c, b):
        pltpu.make_async_copy(
            rows_v.at[b], out_hbm.at[pl.ds(base_w + c * _CH, _CH)],
            wsem[b]).wait()

    for b in range(_NBUF):
        gather(b, b)

    def group(g, carry):
        c0 = g * _NBUF
        for b in range(_NBUF):
            gather_wait(c0 + b, b)
            write(c0 + b, b)

        @pl.when(g + 1 < _NG)
        def _():
            for b in range(_NBUF):
                write_wait(c0 + b, b)
                gather(c0 + _NBUF + b, b)

        return carry

    lax.fori_loop(0, _NG, group, 0)

    for b in range(_NBUF):
        write_wait((_NG - 1) * _NBUF + b, b)


@jax.jit
def kernel(x, cluster_centers):
    xw = x.reshape(_NW, _NCH, _CH)
    out = pl.kernel(
        _gather_body,
        out_type=jax.ShapeDtypeStruct((_N, _D), jnp.float32),
        mesh=plsc.VectorSubcoreMesh(core_axis_name="c", subcore_axis_name="s"),
        compiler_params=pltpu.CompilerParams(use_tc_tiling_on_sc=False),
        scratch_types=[
            pltpu.VMEM((_NCH, _CH), jnp.int32),
            pltpu.VMEM((_NBUF, _CH, _D), jnp.float32),
        ] + [pltpu.SemaphoreType.DMA] * (2 * _NBUF),
    )(xw, cluster_centers)
    return out.reshape(_B, _T, _D)
